# baseline (speedup 1.0000x reference)
import dataclasses
import functools

import jax
import jax.numpy as jnp
from jax import lax
from jax.experimental import pallas as pl
from jax.experimental.pallas import tpu as pltpu
from jax.experimental.pallas import tpu_sc as plsc

NUM_SELECT = 300
B, N, C = 64, 2000, 91
ROW = N * C
CAP1 = 4096
CAP2 = 512
NBINS = 512
W = 36400
NWIN = ROW // W
VPW = W // 16
SAMPLE_RANK = 18


def _prob_body(l_ref, c_ref, i_ref, o_ref):
    o_ref[...] = (
        (jax.nn.sigmoid(l_ref[...]) ** 0.45)
        * (jax.nn.sigmoid(c_ref[...]) ** 0.05)
        * (jax.nn.sigmoid(i_ref[...]) ** 0.5)
    )


def _compute_prob(pred_logits, pred_centers, pred_ious):
    total = B * ROW
    rows = total // 128
    blk = 3640
    prob = pl.pallas_call(
        _prob_body,
        grid=(rows // blk,),
        in_specs=[pl.BlockSpec((blk, 128), lambda i: (i, 0))] * 3,
        out_specs=pl.BlockSpec((blk, 128), lambda i: (i, 0)),
        out_shape=jax.ShapeDtypeStruct((rows, 128), jnp.float32),
    )(
        pred_logits.reshape(rows, 128),
        pred_centers.reshape(rows, 128),
        pred_ious.reshape(rows, 128),
    )
    return prob.reshape(B, ROW)


def _filter_body(prob_hbm, thr_hbm, cv_hbm, ci_hbm, n1_hbm, n2_hbm,
                 win0, win1, thr_v, cand_v, cand_i, hist, c2v, c2i,
                 stage, sem0, sem1, semt):
    wid = lax.axis_index("s") * 2 + lax.axis_index("c")
    iota = lax.broadcasted_iota(jnp.int32, (16,), 0)
    ones = jnp.ones((16,), jnp.int32)

    for rr in range(2):
        r = wid * 2 + rr
        pltpu.async_copy(thr_hbm.at[pl.ds(r * 16, 16)], thr_v, semt).wait()
        thr = thr_v[...]

        @pl.loop(0, NBINS)
        def _(b):
            hist[pl.ds(b * 16, 16)] = jnp.zeros((16,), jnp.int32)

        cps = [None, None]
        rbase = r * ROW
        cps[0] = pltpu.async_copy(prob_hbm.at[pl.ds(rbase, W)], win0, sem0)

        def scan_win(w, buf, off0):
            woff = w * W

            def step(k, off):
                v = buf[pl.ds(k * 16, 16)]
                m = v >= thr
                cnt = jnp.sum(m.astype(jnp.int32))

                @pl.when((cnt > 0) & (off <= CAP1 - 16))
                def _():
                    idxv = iota + (woff + k * 16)
                    plsc.store_compressed(cand_v.at[pl.ds(off, 16)], v, mask=m)
                    plsc.store_compressed(cand_i.at[pl.ds(off, 16)], idxv, mask=m)

                return off + cnt

            return lax.fori_loop(0, VPW, step, off0)

        count = jnp.int32(0)
        for w in range(NWIN):
            buf = win0 if w % 2 == 0 else win1
            nbuf = win1 if w % 2 == 0 else win0
            if w + 1 < NWIN:
                cps[(w + 1) % 2] = pltpu.async_copy(
                    prob_hbm.at[pl.ds(rbase + (w + 1) * W, W)], nbuf,
                    sem1 if (w + 1) % 2 == 1 else sem0)
            cps[w % 2].wait()
            count = scan_win(w, buf, count)

        ncand = jnp.minimum(count, CAP1)

        def hstep(b, _):
            v = cand_v[pl.ds(b * 16, 16)]
            valid = (iota + b * 16) < ncand
            bits = plsc.bitcast(v, jnp.int32)
            binv = jnp.clip(bits >> 21, 0, NBINS - 1)
            plsc.addupdate_scatter(hist, [binv * 16 + iota], ones, mask=valid)
            return 0

        lax.fori_loop(0, CAP1 // 16, hstep, 0)

        def tstep(j, carry):
            cum, t2b = carry
            bidx = NBINS - 1 - j
            s = jnp.sum(hist[pl.ds(bidx * 16, 16)])
            cum2 = cum + s
            found = (cum2 >= NUM_SELECT) & (t2b < 0)
            return cum2, jnp.where(found, bidx, t2b)

        _, t2b = lax.fori_loop(0, NBINS, tstep,
                               (jnp.int32(0), jnp.int32(-1)))
        t2b = jnp.maximum(t2b, 0)
        t2v = plsc.bitcast(
            jnp.broadcast_to(t2b, (16,)).astype(jnp.int32) << 21, jnp.float32)

        @pl.loop(0, CAP2 // 16)
        def _(b):
            c2v[pl.ds(b * 16, 16)] = jnp.full((16,), -1.0, jnp.float32)
            c2i[pl.ds(b * 16, 16)] = jnp.zeros((16,), jnp.int32)

        def rstep(b, off):
            v = cand_v[pl.ds(b * 16, 16)]
            ivx = cand_i[pl.ds(b * 16, 16)]
            m = ((iota + b * 16) < ncand) & (v >= t2v)
            cnt = jnp.sum(m.astype(jnp.int32))

            @pl.when((cnt > 0) & (off <= CAP2 - 16))
            def _():
                plsc.store_compressed(c2v.at[pl.ds(off, 16)], v, mask=m)
                plsc.store_compressed(c2i.at[pl.ds(off, 16)], ivx, mask=m)

            return off + cnt

        count2 = lax.fori_loop(0, CAP1 // 16, rstep, jnp.int32(0))

        pltpu.sync_copy(c2v, cv_hbm.at[pl.ds(r * CAP2, CAP2)])
        pltpu.sync_copy(c2i, ci_hbm.at[pl.ds(r * CAP2, CAP2)])
        stage[...] = jnp.broadcast_to(count, (16,)).astype(jnp.int32)
        pltpu.sync_copy(stage, n1_hbm.at[pl.ds(r * 16, 16)])
        stage[...] = jnp.broadcast_to(count2, (16,)).astype(jnp.int32)
        pltpu.sync_copy(stage, n2_hbm.at[pl.ds(r * 16, 16)])


def _filter_candidates(prob, thr16):
    mesh = plsc.VectorSubcoreMesh(core_axis_name="c", subcore_axis_name="s")
    cp = pltpu.CompilerParams()
    if "needs_layout_passes" in pltpu.CompilerParams.__dataclass_fields__:
        cp = dataclasses.replace(cp, needs_layout_passes=False)
    kern = functools.partial(
        pl.kernel,
        compiler_params=cp,
        out_type=[
            jax.ShapeDtypeStruct((B * CAP2,), jnp.float32),
            jax.ShapeDtypeStruct((B * CAP2,), jnp.int32),
            jax.ShapeDtypeStruct((B * 16,), jnp.int32),
            jax.ShapeDtypeStruct((B * 16,), jnp.int32),
        ],
        mesh=mesh,
        scratch_types=[
            pltpu.VMEM((W,), jnp.float32),
            pltpu.VMEM((W,), jnp.float32),
            pltpu.VMEM((16,), jnp.float32),
            pltpu.VMEM((CAP1,), jnp.float32),
            pltpu.VMEM((CAP1,), jnp.int32),
            pltpu.VMEM((NBINS * 16,), jnp.int32),
            pltpu.VMEM((CAP2,), jnp.float32),
            pltpu.VMEM((CAP2,), jnp.int32),
            pltpu.VMEM((16,), jnp.int32),
            pltpu.SemaphoreType.DMA,
            pltpu.SemaphoreType.DMA,
            pltpu.SemaphoreType.DMA,
        ],
    )(_filter_body)
    cv, ci, n1, n2 = kern(prob.reshape(B * ROW), thr16.reshape(B * 16))
    return (cv.reshape(B, CAP2), ci.reshape(B, CAP2),
            n1.reshape(B, 16), n2.reshape(B, 16))


def kernel(pred_logits, pred_boxes, pred_centers, pred_ious, target_sizes, img_metas):
    prob = _compute_prob(pred_logits, pred_centers, pred_ious)

    sample = prob[:, ::91]
    thr = lax.top_k(sample, SAMPLE_RANK)[0][:, SAMPLE_RANK - 1]
    thr16 = jnp.broadcast_to(thr[:, None], (B, 16))

    cv, ci, n1, n2 = _filter_candidates(prob, thr16)
    ok = jnp.all((n1[:, 0] >= NUM_SELECT) & (n1[:, 0] <= CAP1)
                 & (n2[:, 0] >= NUM_SELECT) & (n2[:, 0] <= CAP2))

    img_h = target_sizes[:, 0].astype(jnp.float32)
    img_w = target_sizes[:, 1].astype(jnp.float32)
    scale_fct = jnp.stack([img_w, img_h, img_w, img_h], axis=1)
    cx, cy, w_, h_ = (pred_boxes[..., k] for k in range(4))
    boxes_all = jnp.stack(
        [cx - 0.5 * w_, cy - 0.5 * h_, cx + 0.5 * w_, cy + 0.5 * h_], axis=-1)

    def finish(topk_values, topk_indexes):
        scores = topk_values
        topk_boxes = topk_indexes // C
        labels = topk_indexes % C
        boxes = jnp.take_along_axis(boxes_all, topk_boxes[:, :, None], axis=1)
        boxes = boxes * scale_fct[:, None, :]
        return boxes, scores, labels

    def fast():
        v300, pos = lax.top_k(cv, NUM_SELECT)
        i300 = jnp.take_along_axis(ci, pos, axis=1)
        return finish(v300, i300)

    def slow():
        return finish(*lax.top_k(prob, NUM_SELECT))

    return lax.cond(ok, fast, slow)


# speedup vs baseline: 4.4256x; 4.4256x over previous
import dataclasses
import functools

import jax
import jax.numpy as jnp
from jax import lax
from jax.experimental import pallas as pl
from jax.experimental.pallas import tpu as pltpu
from jax.experimental.pallas import tpu_sc as plsc

NUM_SELECT = 300
B, N, C = 64, 2000, 91
ROW = N * C
CAP1 = 4096
CAP2 = 512
NBINS = 512
W = 36400
NWIN = ROW // W
VPW = W // 16
SAMPLE_RANK = 18


def _prob_body(l_ref, c_ref, i_ref, o_ref):
    o_ref[...] = (
        (jax.nn.sigmoid(l_ref[...]) ** 0.45)
        * (jax.nn.sigmoid(c_ref[...]) ** 0.05)
        * (jax.nn.sigmoid(i_ref[...]) ** 0.5)
    )


def _compute_prob(pred_logits, pred_centers, pred_ious):
    total = B * ROW
    rows = total // 128
    blk = 3640
    prob = pl.pallas_call(
        _prob_body,
        grid=(rows // blk,),
        in_specs=[pl.BlockSpec((blk, 128), lambda i: (i, 0))] * 3,
        out_specs=pl.BlockSpec((blk, 128), lambda i: (i, 0)),
        out_shape=jax.ShapeDtypeStruct((rows, 128), jnp.float32),
    )(
        pred_logits.reshape(rows, 128),
        pred_centers.reshape(rows, 128),
        pred_ious.reshape(rows, 128),
    )
    return prob.reshape(B, ROW)


def _filter_body(prob_hbm, thr_hbm, cv_hbm, ci_hbm, n1_hbm, n2_hbm,
                 win0, win1, thr_v, cand_v, cand_i, hist, c2v, c2i,
                 stage, sem0, sem1, semt):
    wid = lax.axis_index("s") * 2 + lax.axis_index("c")
    iota = lax.broadcasted_iota(jnp.int32, (16,), 0)
    ones = jnp.ones((16,), jnp.int32)

    for rr in range(2):
        r = wid * 2 + rr
        pltpu.async_copy(thr_hbm.at[pl.ds(r * 16, 16)], thr_v, semt).wait()
        thr = thr_v[...]

        @pl.loop(0, NBINS)
        def _(b):
            hist[pl.ds(b * 16, 16)] = jnp.zeros((16,), jnp.int32)

        cps = [None, None]
        rbase = r * ROW
        cps[0] = pltpu.async_copy(prob_hbm.at[pl.ds(rbase, W)], win0, sem0)

        def scan_win(w, buf, off0):
            woff = w * W

            def step(k, off):
                v = buf[pl.ds(k * 16, 16)]
                m = v >= thr
                cnt = jnp.sum(m.astype(jnp.int32))

                @pl.when((cnt > 0) & (off <= CAP1 - 16))
                def _():
                    idxv = iota + (woff + k * 16)
                    plsc.store_compressed(cand_v.at[pl.ds(off, 16)], v, mask=m)
                    plsc.store_compressed(cand_i.at[pl.ds(off, 16)], idxv, mask=m)

                return off + cnt

            return lax.fori_loop(0, VPW, step, off0)

        count = jnp.int32(0)
        for w in range(NWIN):
            buf = win0 if w % 2 == 0 else win1
            nbuf = win1 if w % 2 == 0 else win0
            if w + 1 < NWIN:
                cps[(w + 1) % 2] = pltpu.async_copy(
                    prob_hbm.at[pl.ds(rbase + (w + 1) * W, W)], nbuf,
                    sem1 if (w + 1) % 2 == 1 else sem0)
            cps[w % 2].wait()
            count = scan_win(w, buf, count)

        ncand = jnp.minimum(count, CAP1)

        scale = 512.0 / jnp.maximum(1.0 - thr, 1e-30)

        def binf(v):
            return jnp.clip(((v - thr) * scale).astype(jnp.int32), 0, NBINS - 1)

        def hstep(b, _):
            v = cand_v[pl.ds(b * 16, 16)]
            valid = (iota + b * 16) < ncand
            plsc.addupdate_scatter(hist, [binf(v) * 16 + iota], ones, mask=valid)
            return 0

        lax.fori_loop(0, CAP1 // 16, hstep, 0)

        def tstep(j, carry):
            cum, t2b = carry
            bidx = NBINS - 1 - j
            s = jnp.sum(hist[pl.ds(bidx * 16, 16)])
            cum2 = cum + s
            found = (cum2 >= NUM_SELECT) & (t2b < 0)
            return cum2, jnp.where(found, bidx, t2b)

        _, t2b = lax.fori_loop(0, NBINS, tstep,
                               (jnp.int32(0), jnp.int32(-1)))
        t2b = jnp.maximum(t2b, 0)
        t2bv = jnp.broadcast_to(t2b, (16,)).astype(jnp.int32)

        @pl.loop(0, CAP2 // 16)
        def _(b):
            c2v[pl.ds(b * 16, 16)] = jnp.full((16,), -1.0, jnp.float32)
            c2i[pl.ds(b * 16, 16)] = jnp.zeros((16,), jnp.int32)

        def rstep(b, off):
            v = cand_v[pl.ds(b * 16, 16)]
            ivx = cand_i[pl.ds(b * 16, 16)]
            m = ((iota + b * 16) < ncand) & (binf(v) >= t2bv)
            cnt = jnp.sum(m.astype(jnp.int32))

            @pl.when((cnt > 0) & (off <= CAP2 - 16))
            def _():
                plsc.store_compressed(c2v.at[pl.ds(off, 16)], v, mask=m)
                plsc.store_compressed(c2i.at[pl.ds(off, 16)], ivx, mask=m)

            return off + cnt

        count2 = lax.fori_loop(0, CAP1 // 16, rstep, jnp.int32(0))

        pltpu.sync_copy(c2v, cv_hbm.at[pl.ds(r * CAP2, CAP2)])
        pltpu.sync_copy(c2i, ci_hbm.at[pl.ds(r * CAP2, CAP2)])
        stage[...] = jnp.broadcast_to(count, (16,)).astype(jnp.int32)
        pltpu.sync_copy(stage, n1_hbm.at[pl.ds(r * 16, 16)])
        stage[...] = jnp.broadcast_to(count2, (16,)).astype(jnp.int32)
        pltpu.sync_copy(stage, n2_hbm.at[pl.ds(r * 16, 16)])


def _filter_candidates(prob, thr16):
    mesh = plsc.VectorSubcoreMesh(core_axis_name="c", subcore_axis_name="s")
    cp = pltpu.CompilerParams()
    if "needs_layout_passes" in pltpu.CompilerParams.__dataclass_fields__:
        cp = dataclasses.replace(cp, needs_layout_passes=False)
    kern = functools.partial(
        pl.kernel,
        compiler_params=cp,
        out_type=[
            jax.ShapeDtypeStruct((B * CAP2,), jnp.float32),
            jax.ShapeDtypeStruct((B * CAP2,), jnp.int32),
            jax.ShapeDtypeStruct((B * 16,), jnp.int32),
            jax.ShapeDtypeStruct((B * 16,), jnp.int32),
        ],
        mesh=mesh,
        scratch_types=[
            pltpu.VMEM((W,), jnp.float32),
            pltpu.VMEM((W,), jnp.float32),
            pltpu.VMEM((16,), jnp.float32),
            pltpu.VMEM((CAP1,), jnp.float32),
            pltpu.VMEM((CAP1,), jnp.int32),
            pltpu.VMEM((NBINS * 16,), jnp.int32),
            pltpu.VMEM((CAP2,), jnp.float32),
            pltpu.VMEM((CAP2,), jnp.int32),
            pltpu.VMEM((16,), jnp.int32),
            pltpu.SemaphoreType.DMA,
            pltpu.SemaphoreType.DMA,
            pltpu.SemaphoreType.DMA,
        ],
    )(_filter_body)
    cv, ci, n1, n2 = kern(prob.reshape(B * ROW), thr16.reshape(B * 16))
    return (cv.reshape(B, CAP2), ci.reshape(B, CAP2),
            n1.reshape(B, 16), n2.reshape(B, 16))


def kernel(pred_logits, pred_boxes, pred_centers, pred_ious, target_sizes, img_metas):
    prob = _compute_prob(pred_logits, pred_centers, pred_ious)

    sample = prob[:, ::91]
    thr = lax.top_k(sample, SAMPLE_RANK)[0][:, SAMPLE_RANK - 1]
    thr16 = jnp.broadcast_to(thr[:, None], (B, 16))

    cv, ci, n1, n2 = _filter_candidates(prob, thr16)
    ok = jnp.all((n1[:, 0] >= NUM_SELECT) & (n1[:, 0] <= CAP1)
                 & (n2[:, 0] >= NUM_SELECT) & (n2[:, 0] <= CAP2))

    img_h = target_sizes[:, 0].astype(jnp.float32)
    img_w = target_sizes[:, 1].astype(jnp.float32)
    scale_fct = jnp.stack([img_w, img_h, img_w, img_h], axis=1)
    cx, cy, w_, h_ = (pred_boxes[..., k] for k in range(4))
    boxes_all = jnp.stack(
        [cx - 0.5 * w_, cy - 0.5 * h_, cx + 0.5 * w_, cy + 0.5 * h_], axis=-1)

    def finish(topk_values, topk_indexes):
        scores = topk_values
        topk_boxes = topk_indexes // C
        labels = topk_indexes % C
        boxes = jnp.take_along_axis(boxes_all, topk_boxes[:, :, None], axis=1)
        boxes = boxes * scale_fct[:, None, :]
        return boxes, scores, labels

    def fast():
        v300, pos = lax.top_k(cv, NUM_SELECT)
        i300 = jnp.take_along_axis(ci, pos, axis=1)
        return finish(v300, i300)

    def slow():
        return finish(*lax.top_k(prob, NUM_SELECT))

    return lax.cond(ok, fast, slow)


# speedup vs baseline: 6.2870x; 1.4206x over previous
import dataclasses
import functools

import jax
import jax.numpy as jnp
from jax import lax
from jax.experimental import pallas as pl
from jax.experimental.pallas import tpu as pltpu
from jax.experimental.pallas import tpu_sc as plsc

NUM_SELECT = 300
B, N, C = 64, 2000, 91
PROW = N * 128
CAP1 = 4096
CAP2 = 512
NBINS = 512
WROWS = 200
NWIN = N // WROWS
SAMPLE_RANK = 18


def _prob_body(l_ref, c_ref, i_ref, o_ref):
    p = (
        (jax.nn.sigmoid(l_ref[...]) ** 0.45)
        * (jax.nn.sigmoid(c_ref[...]) ** 0.05)
        * (jax.nn.sigmoid(i_ref[...]) ** 0.5)
    )
    o_ref[...] = jnp.full(o_ref.shape, -1.0, jnp.float32)
    o_ref[:, pl.ds(0, C)] = p


def _compute_prob(pred_logits, pred_centers, pred_ious):
    rows = B * N
    blk = 5120
    return pl.pallas_call(
        _prob_body,
        grid=(rows // blk,),
        in_specs=[pl.BlockSpec((blk, C), lambda i: (i, 0))] * 3,
        out_specs=pl.BlockSpec((blk, 128), lambda i: (i, 0)),
        out_shape=jax.ShapeDtypeStruct((rows, 128), jnp.float32),
    )(
        pred_logits.reshape(rows, C),
        pred_centers.reshape(rows, C),
        pred_ious.reshape(rows, C),
    )


def _filter_body(prob_hbm, thr_hbm, cv_hbm, ci_hbm, n1_hbm, n2_hbm,
                 win0, win1, thr_v, cand_v, cand_i, hist, c2v, c2i,
                 stage, sem0, sem1, semt):
    wid = lax.axis_index("s") * 2 + lax.axis_index("c")
    iota = lax.broadcasted_iota(jnp.int32, (16,), 0)
    ones = jnp.ones((16,), jnp.int32)

    for rr in range(2):
        r = wid * 2 + rr
        pltpu.async_copy(thr_hbm.at[pl.ds(r * 16, 16)], thr_v, semt).wait()
        thr = thr_v[...]

        @pl.loop(0, NBINS)
        def _(b):
            hist[pl.ds(b * 16, 16)] = jnp.zeros((16,), jnp.int32)

        def scan_win(w, buf, off0):
            brow = w * WROWS

            def step(kr, off):
                vb = (brow + kr) * 128
                for l in range(8):
                    v = buf[kr, pl.ds(l * 16, 16)]
                    m = v >= thr
                    cnt = jnp.sum(m.astype(jnp.int32))

                    @pl.when((cnt > 0) & (off <= CAP1 - 16))
                    def _(v=v, m=m, off=off, l=l):
                        idxv = iota + (vb + l * 16)
                        plsc.store_compressed(
                            cand_v.at[pl.ds(off, 16)], v, mask=m)
                        plsc.store_compressed(
                            cand_i.at[pl.ds(off, 16)], idxv, mask=m)

                    off = off + cnt
                return off

            return lax.fori_loop(0, WROWS, step, off0)

        cps = [None, None]
        cps[0] = pltpu.async_copy(prob_hbm.at[pl.ds(r * N, WROWS)], win0, sem0)
        count = jnp.int32(0)
        for w in range(NWIN):
            buf = win0 if w % 2 == 0 else win1
            nbuf = win1 if w % 2 == 0 else win0
            if w + 1 < NWIN:
                cps[(w + 1) % 2] = pltpu.async_copy(
                    prob_hbm.at[pl.ds(r * N + (w + 1) * WROWS, WROWS)], nbuf,
                    sem1 if (w + 1) % 2 == 1 else sem0)
            cps[w % 2].wait()
            count = scan_win(w, buf, count)

        ncand = jnp.minimum(count, CAP1)

        scale = 512.0 / jnp.maximum(1.0 - thr, 1e-30)

        def binf(v):
            return jnp.clip(((v - thr) * scale).astype(jnp.int32), 0, NBINS - 1)

        def hstep(b, _):
            v = cand_v[pl.ds(b * 16, 16)]
            valid = (iota + b * 16) < ncand
            plsc.addupdate_scatter(hist, [binf(v) * 16 + iota], ones, mask=valid)
            return 0

        lax.fori_loop(0, CAP1 // 16, hstep, 0)

        def tstep(j, carry):
            cum, t2b = carry
            bidx = NBINS - 1 - j
            s = jnp.sum(hist[pl.ds(bidx * 16, 16)])
            cum2 = cum + s
            found = (cum2 >= NUM_SELECT) & (t2b < 0)
            return cum2, jnp.where(found, bidx, t2b)

        _, t2b = lax.fori_loop(0, NBINS, tstep,
                               (jnp.int32(0), jnp.int32(-1)))
        t2b = jnp.maximum(t2b, 0)
        t2bv = jnp.broadcast_to(t2b, (16,)).astype(jnp.int32)

        @pl.loop(0, CAP2 // 16)
        def _(b):
            c2v[pl.ds(b * 16, 16)] = jnp.full((16,), -1.0, jnp.float32)
            c2i[pl.ds(b * 16, 16)] = jnp.zeros((16,), jnp.int32)

        def rstep(b, off):
            v = cand_v[pl.ds(b * 16, 16)]
            ivx = cand_i[pl.ds(b * 16, 16)]
            m = ((iota + b * 16) < ncand) & (binf(v) >= t2bv)
            cnt = jnp.sum(m.astype(jnp.int32))

            @pl.when((cnt > 0) & (off <= CAP2 - 16))
            def _():
                plsc.store_compressed(c2v.at[pl.ds(off, 16)], v, mask=m)
                plsc.store_compressed(c2i.at[pl.ds(off, 16)], ivx, mask=m)

            return off + cnt

        count2 = lax.fori_loop(0, CAP1 // 16, rstep, jnp.int32(0))

        pltpu.sync_copy(c2v, cv_hbm.at[pl.ds(r * CAP2, CAP2)])
        pltpu.sync_copy(c2i, ci_hbm.at[pl.ds(r * CAP2, CAP2)])
        stage[...] = jnp.broadcast_to(count, (16,)).astype(jnp.int32)
        pltpu.sync_copy(stage, n1_hbm.at[pl.ds(r * 16, 16)])
        stage[...] = jnp.broadcast_to(count2, (16,)).astype(jnp.int32)
        pltpu.sync_copy(stage, n2_hbm.at[pl.ds(r * 16, 16)])


def _filter_candidates(prob2d, thr16):
    mesh = plsc.VectorSubcoreMesh(core_axis_name="c", subcore_axis_name="s")
    cp = pltpu.CompilerParams()
    if "needs_layout_passes" in pltpu.CompilerParams.__dataclass_fields__:
        cp = dataclasses.replace(cp, needs_layout_passes=False)
    kern = functools.partial(
        pl.kernel,
        compiler_params=cp,
        out_type=[
            jax.ShapeDtypeStruct((B * CAP2,), jnp.float32),
            jax.ShapeDtypeStruct((B * CAP2,), jnp.int32),
            jax.ShapeDtypeStruct((B * 16,), jnp.int32),
            jax.ShapeDtypeStruct((B * 16,), jnp.int32),
        ],
        mesh=mesh,
        scratch_types=[
            pltpu.VMEM((WROWS, 128), jnp.float32),
            pltpu.VMEM((WROWS, 128), jnp.float32),
            pltpu.VMEM((16,), jnp.float32),
            pltpu.VMEM((CAP1,), jnp.float32),
            pltpu.VMEM((CAP1,), jnp.int32),
            pltpu.VMEM((NBINS * 16,), jnp.int32),
            pltpu.VMEM((CAP2,), jnp.float32),
            pltpu.VMEM((CAP2,), jnp.int32),
            pltpu.VMEM((16,), jnp.int32),
            pltpu.SemaphoreType.DMA,
            pltpu.SemaphoreType.DMA,
            pltpu.SemaphoreType.DMA,
        ],
    )(_filter_body)
    cv, ci, n1, n2 = kern(prob2d, thr16.reshape(B * 16))
    return (cv.reshape(B, CAP2), ci.reshape(B, CAP2),
            n1.reshape(B, 16), n2.reshape(B, 16))


def kernel(pred_logits, pred_boxes, pred_centers, pred_ious, target_sizes, img_metas):
    prob2d = _compute_prob(pred_logits, pred_centers, pred_ious)

    sample = prob2d[:, 0].reshape(B, N)
    thr = lax.top_k(sample, SAMPLE_RANK)[0][:, SAMPLE_RANK - 1]
    thr16 = jnp.broadcast_to(thr[:, None], (B, 16))

    cv, ci, n1, n2 = _filter_candidates(prob2d, thr16)
    ok = jnp.all((n1[:, 0] >= NUM_SELECT) & (n1[:, 0] <= CAP1)
                 & (n2[:, 0] >= NUM_SELECT) & (n2[:, 0] <= CAP2))

    img_h = target_sizes[:, 0].astype(jnp.float32)
    img_w = target_sizes[:, 1].astype(jnp.float32)
    scale_fct = jnp.stack([img_w, img_h, img_w, img_h], axis=1)
    cx, cy, w_, h_ = (pred_boxes[..., k] for k in range(4))
    boxes_all = jnp.stack(
        [cx - 0.5 * w_, cy - 0.5 * h_, cx + 0.5 * w_, cy + 0.5 * h_], axis=-1)

    def finish(topk_values, padded_indexes):
        scores = topk_values
        topk_boxes = padded_indexes >> 7
        labels = padded_indexes & 127
        boxes = jnp.take_along_axis(boxes_all, topk_boxes[:, :, None], axis=1)
        boxes = boxes * scale_fct[:, None, :]
        return boxes, scores, labels

    def fast():
        v300, pos = lax.top_k(cv, NUM_SELECT)
        i300 = jnp.take_along_axis(ci, pos, axis=1)
        return finish(v300, i300)

    def slow():
        return finish(*lax.top_k(prob2d.reshape(B, PROW), NUM_SELECT))

    return lax.cond(ok, fast, slow)


# speedup vs baseline: 6.8006x; 1.0817x over previous
import dataclasses
import functools

import jax
import jax.numpy as jnp
from jax import lax
from jax.experimental import pallas as pl
from jax.experimental.pallas import tpu as pltpu
from jax.experimental.pallas import tpu_sc as plsc

NUM_SELECT = 300
B, N, C = 64, 2000, 91
PROW = N * 128
CAP1 = 4096
CAP2 = 512
NBINS = 512
WROWS = 200
NWIN = N // WROWS
SAMPLE_RANK = 18


def _prob_body(l_ref, c_ref, i_ref, o_ref):
    p = (
        (jax.nn.sigmoid(l_ref[...]) ** 0.45)
        * (jax.nn.sigmoid(c_ref[...]) ** 0.05)
        * (jax.nn.sigmoid(i_ref[...]) ** 0.5)
    )
    o_ref[...] = jnp.full(o_ref.shape, -1.0, jnp.float32)
    o_ref[:, pl.ds(0, C)] = p


def _compute_prob(pred_logits, pred_centers, pred_ious):
    rows = B * N
    blk = 5120
    return pl.pallas_call(
        _prob_body,
        grid=(rows // blk,),
        in_specs=[pl.BlockSpec((blk, C), lambda i: (i, 0))] * 3,
        out_specs=pl.BlockSpec((blk, 128), lambda i: (i, 0)),
        out_shape=jax.ShapeDtypeStruct((rows, 128), jnp.float32),
    )(
        pred_logits.reshape(rows, C),
        pred_centers.reshape(rows, C),
        pred_ious.reshape(rows, C),
    )


def _filter_body(prob_hbm, thr_hbm, cv_hbm, ci_hbm, n1_hbm, n2_hbm,
                 win0, win1, thr_v, cand_v, cand_i, hist, c2v, c2i,
                 stage, sem0, sem1, semt):
    wid = lax.axis_index("s") * 2 + lax.axis_index("c")
    iota = lax.broadcasted_iota(jnp.int32, (16,), 0)
    ones = jnp.ones((16,), jnp.int32)

    def popcnt(m):
        c = plsc.all_reduce_population_count(m)
        return lax.squeeze(lax.slice(c, (0,), (1,)), (0,))

    for rr in range(2):
        r = wid * 2 + rr
        pltpu.async_copy(thr_hbm.at[pl.ds(r * 16, 16)], thr_v, semt).wait()
        thr = thr_v[...]

        @pl.loop(0, NBINS)
        def _(b):
            hist[pl.ds(b * 16, 16)] = jnp.zeros((16,), jnp.int32)

        def scan_win(w, buf, off0):
            brow = w * WROWS

            def step(kr, off):
                vb = (brow + kr) * 128
                for l in range(8):
                    v = buf[kr, pl.ds(l * 16, 16)]
                    m = v >= thr
                    cnt = popcnt(m)

                    @pl.when((cnt > 0) & (off <= CAP1 - 16))
                    def _(v=v, m=m, off=off, l=l):
                        idxv = iota + (vb + l * 16)
                        plsc.store_compressed(
                            cand_v.at[pl.ds(off, 16)], v, mask=m)
                        plsc.store_compressed(
                            cand_i.at[pl.ds(off, 16)], idxv, mask=m)

                    off = off + cnt
                return off

            return lax.fori_loop(0, WROWS, step, off0)

        cps = [None, None]
        cps[0] = pltpu.async_copy(prob_hbm.at[pl.ds(r * N, WROWS)], win0, sem0)
        count = jnp.int32(0)
        for w in range(NWIN):
            buf = win0 if w % 2 == 0 else win1
            nbuf = win1 if w % 2 == 0 else win0
            if w + 1 < NWIN:
                cps[(w + 1) % 2] = pltpu.async_copy(
                    prob_hbm.at[pl.ds(r * N + (w + 1) * WROWS, WROWS)], nbuf,
                    sem1 if (w + 1) % 2 == 1 else sem0)
            cps[w % 2].wait()
            count = scan_win(w, buf, count)

        ncand = jnp.minimum(count, CAP1)

        scale = 512.0 / jnp.maximum(1.0 - thr, 1e-30)

        def binf(v):
            return jnp.clip(((v - thr) * scale).astype(jnp.int32), 0, NBINS - 1)

        def hstep(b, _):
            v = cand_v[pl.ds(b * 16, 16)]
            valid = (iota + b * 16) < ncand
            plsc.addupdate_scatter(hist, [binf(v) * 16 + iota], ones, mask=valid)
            return 0

        lax.fori_loop(0, CAP1 // 16, hstep, 0)

        def tstep(j, carry):
            cum, t2b = carry
            bidx = NBINS - 1 - j
            s = jnp.sum(hist[pl.ds(bidx * 16, 16)])
            cum2 = cum + s
            found = (cum2 >= NUM_SELECT) & (t2b < 0)
            return cum2, jnp.where(found, bidx, t2b)

        _, t2b = lax.fori_loop(0, NBINS, tstep,
                               (jnp.int32(0), jnp.int32(-1)))
        t2b = jnp.maximum(t2b, 0)
        t2bv = jnp.broadcast_to(t2b, (16,)).astype(jnp.int32)

        @pl.loop(0, CAP2 // 16)
        def _(b):
            c2v[pl.ds(b * 16, 16)] = jnp.full((16,), -1.0, jnp.float32)
            c2i[pl.ds(b * 16, 16)] = jnp.zeros((16,), jnp.int32)

        def rstep(b, off):
            v = cand_v[pl.ds(b * 16, 16)]
            ivx = cand_i[pl.ds(b * 16, 16)]
            m = ((iota + b * 16) < ncand) & (binf(v) >= t2bv)
            cnt = popcnt(m)

            @pl.when((cnt > 0) & (off <= CAP2 - 16))
            def _():
                plsc.store_compressed(c2v.at[pl.ds(off, 16)], v, mask=m)
                plsc.store_compressed(c2i.at[pl.ds(off, 16)], ivx, mask=m)

            return off + cnt

        count2 = lax.fori_loop(0, CAP1 // 16, rstep, jnp.int32(0))

        pltpu.sync_copy(c2v, cv_hbm.at[pl.ds(r * CAP2, CAP2)])
        pltpu.sync_copy(c2i, ci_hbm.at[pl.ds(r * CAP2, CAP2)])
        stage[...] = jnp.broadcast_to(count, (16,)).astype(jnp.int32)
        pltpu.sync_copy(stage, n1_hbm.at[pl.ds(r * 16, 16)])
        stage[...] = jnp.broadcast_to(count2, (16,)).astype(jnp.int32)
        pltpu.sync_copy(stage, n2_hbm.at[pl.ds(r * 16, 16)])


def _filter_candidates(prob2d, thr16):
    mesh = plsc.VectorSubcoreMesh(core_axis_name="c", subcore_axis_name="s")
    cp = pltpu.CompilerParams()
    if "needs_layout_passes" in pltpu.CompilerParams.__dataclass_fields__:
        cp = dataclasses.replace(cp, needs_layout_passes=False)
    kern = functools.partial(
        pl.kernel,
        compiler_params=cp,
        out_type=[
            jax.ShapeDtypeStruct((B * CAP2,), jnp.float32),
            jax.ShapeDtypeStruct((B * CAP2,), jnp.int32),
            jax.ShapeDtypeStruct((B * 16,), jnp.int32),
            jax.ShapeDtypeStruct((B * 16,), jnp.int32),
        ],
        mesh=mesh,
        scratch_types=[
            pltpu.VMEM((WROWS, 128), jnp.float32),
            pltpu.VMEM((WROWS, 128), jnp.float32),
            pltpu.VMEM((16,), jnp.float32),
            pltpu.VMEM((CAP1,), jnp.float32),
            pltpu.VMEM((CAP1,), jnp.int32),
            pltpu.VMEM((NBINS * 16,), jnp.int32),
            pltpu.VMEM((CAP2,), jnp.float32),
            pltpu.VMEM((CAP2,), jnp.int32),
            pltpu.VMEM((16,), jnp.int32),
            pltpu.SemaphoreType.DMA,
            pltpu.SemaphoreType.DMA,
            pltpu.SemaphoreType.DMA,
        ],
    )(_filter_body)
    cv, ci, n1, n2 = kern(prob2d, thr16.reshape(B * 16))
    return (cv.reshape(B, CAP2), ci.reshape(B, CAP2),
            n1.reshape(B, 16), n2.reshape(B, 16))


def kernel(pred_logits, pred_boxes, pred_centers, pred_ious, target_sizes, img_metas):
    prob2d = _compute_prob(pred_logits, pred_centers, pred_ious)

    sample = prob2d[:, 0].reshape(B, N)
    thr = lax.top_k(sample, SAMPLE_RANK)[0][:, SAMPLE_RANK - 1]
    thr16 = jnp.broadcast_to(thr[:, None], (B, 16))

    cv, ci, n1, n2 = _filter_candidates(prob2d, thr16)
    ok = jnp.all((n1[:, 0] >= NUM_SELECT) & (n1[:, 0] <= CAP1)
                 & (n2[:, 0] >= NUM_SELECT) & (n2[:, 0] <= CAP2))

    img_h = target_sizes[:, 0].astype(jnp.float32)
    img_w = target_sizes[:, 1].astype(jnp.float32)
    scale_fct = jnp.stack([img_w, img_h, img_w, img_h], axis=1)
    cx, cy, w_, h_ = (pred_boxes[..., k] for k in range(4))
    boxes_all = jnp.stack(
        [cx - 0.5 * w_, cy - 0.5 * h_, cx + 0.5 * w_, cy + 0.5 * h_], axis=-1)

    def finish(topk_values, padded_indexes):
        scores = topk_values
        topk_boxes = padded_indexes >> 7
        labels = padded_indexes & 127
        boxes = jnp.take_along_axis(boxes_all, topk_boxes[:, :, None], axis=1)
        boxes = boxes * scale_fct[:, None, :]
        return boxes, scores, labels

    def fast():
        v300, pos = lax.top_k(cv, NUM_SELECT)
        i300 = jnp.take_along_axis(ci, pos, axis=1)
        return finish(v300, i300)

    def slow():
        return finish(*lax.top_k(prob2d.reshape(B, PROW), NUM_SELECT))

    return lax.cond(ok, fast, slow)
